# baseline (speedup 1.0000x reference)
import jax
import jax.numpy as jnp
from jax.experimental import pallas as pl
from jax.experimental.pallas import tpu as pltpu


def _gate_pool_conv_kernel(x_ref, dct_ref, w1_ref, w2_ref, band_ref, wb_ref,
                           att_ref, conv_ref, parts_ref):
    C, H, W = x_ref.shape[1], x_ref.shape[2], x_ref.shape[3]
    x = x_ref[0]

    prod = x * dct_ref[...]
    y = jnp.sum(jnp.sum(prod, axis=2), axis=1)[None, :]
    h = jnp.maximum(jnp.dot(y, w1_ref[...], preferred_element_type=jnp.float32), 0.0)
    att = jax.nn.sigmoid(jnp.dot(h, w2_ref[...], preferred_element_type=jnp.float32))
    att_ref[0] = att

    xs = x * att[0][:, None, None]
    mx = jnp.max(xs, axis=0)
    mn = jnp.sum(xs, axis=0) * (1.0 / C)

    zr = jnp.zeros((3, W), jnp.float32)
    zc = jnp.zeros((H + 6, 3), jnp.float32)
    cols = []
    for p in (mx, mn):
        p = jnp.concatenate([zr, p, zr], axis=0)
        p = jnp.concatenate([zc, p, zc], axis=1)
        for dx in range(7):
            cols.append(p[:, dx:dx + W])
    cols = jnp.concatenate(cols, axis=0)
    acc = jnp.dot(band_ref[...], cols,
                  preferred_element_type=jnp.float32) + wb_ref[98]
    conv_ref[0] = acc

    s1 = jnp.sum(acc)
    s2 = jnp.sum(acc * acc)
    lane = jax.lax.broadcasted_iota(jnp.int32, (128,), 0)
    parts_ref[0, 0] = jnp.where(lane == 0, s1, 0.0) + jnp.where(lane == 1, s2, 0.0)


def _apply_kernel(x_ref, att_ref, conv_ref, parts_ref, out_ref):
    n = pl.program_id(0)
    C, H, W = x_ref.shape[1], x_ref.shape[2], x_ref.shape[3]
    N = conv_ref.shape[0]
    numel = N * H * W

    lane = jax.lax.broadcasted_iota(jnp.int32, (128,), 0)
    tot = jnp.sum(parts_ref[:, 0, :], axis=0)
    s1 = jnp.sum(jnp.where(lane == 0, tot, 0.0))
    s2 = jnp.sum(jnp.where(lane == 1, tot, 0.0))
    mean = s1 * (1.0 / numel)
    var = (s2 - s1 * mean) * (1.0 / (numel - 1))
    inv_sigma = 1.0 / (jnp.sqrt(2.0 * jnp.pi) * jnp.sqrt(var))

    d = conv_ref[n] - mean
    scale = jnp.exp(-(d * d) / (2.0 * var)) * inv_sigma

    for c in range(C):
        out_ref[0, c] = x_ref[0, c] * (att_ref[n, c] * scale)


def kernel(x, dct_w, w1, w2, conv_wb):
    N, C, H, W = x.shape
    Cr = w1.shape[1]

    wt = conv_wb[:98].reshape(2, 7, 7)
    eyes = jnp.stack([jnp.eye(H, H + 6, k=dy, dtype=jnp.float32)
                      for dy in range(7)])
    blocks = [jnp.einsum("y,yij->ij", wt[c, :, dx], eyes)
              for c in range(2) for dx in range(7)]
    band = jnp.concatenate(blocks, axis=1)

    att3, conv, parts = pl.pallas_call(
        _gate_pool_conv_kernel,
        grid=(N,),
        in_specs=[
            pl.BlockSpec((1, C, H, W), lambda n: (n, 0, 0, 0)),
            pl.BlockSpec((C, H, W), lambda n: (0, 0, 0)),
            pl.BlockSpec((C, Cr), lambda n: (0, 0)),
            pl.BlockSpec((Cr, C), lambda n: (0, 0)),
            pl.BlockSpec((H, 14 * (H + 6)), lambda n: (0, 0)),
            pl.BlockSpec(memory_space=pltpu.MemorySpace.SMEM),
        ],
        out_specs=(
            pl.BlockSpec((1, 1, C), lambda n: (n, 0, 0)),
            pl.BlockSpec((1, H, W), lambda n: (n, 0, 0)),
            pl.BlockSpec((1, 1, 128), lambda n: (n, 0, 0)),
        ),
        out_shape=(
            jax.ShapeDtypeStruct((N, 1, C), jnp.float32),
            jax.ShapeDtypeStruct((N, H, W), jnp.float32),
            jax.ShapeDtypeStruct((N, 1, 128), jnp.float32),
        ),
        compiler_params=pltpu.CompilerParams(dimension_semantics=("parallel",)),
    )(x, dct_w, w1, w2, band, conv_wb)

    out = pl.pallas_call(
        _apply_kernel,
        grid=(N,),
        in_specs=[
            pl.BlockSpec((1, C, H, W), lambda n: (n, 0, 0, 0)),
            pl.BlockSpec(memory_space=pltpu.MemorySpace.SMEM),
            pl.BlockSpec((N, H, W), lambda n: (0, 0, 0)),
            pl.BlockSpec((N, 1, 128), lambda n: (0, 0, 0)),
        ],
        out_specs=pl.BlockSpec((1, C, H, W), lambda n: (n, 0, 0, 0)),
        out_shape=jax.ShapeDtypeStruct((N, C, H, W), jnp.float32),
        compiler_params=pltpu.CompilerParams(dimension_semantics=("parallel",)),
    )(x, att3.reshape(N, C), conv, parts)

    return out


# speedup vs baseline: 1.0861x; 1.0861x over previous
import jax
import jax.numpy as jnp
from jax.experimental import pallas as pl
from jax.experimental.pallas import tpu as pltpu

_BB = 4


def _gate_pool_conv_kernel(x_ref, dct_ref, w1_ref, w2_ref, band_ref, wb_ref,
                           att_ref, conv_ref, parts_ref):
    BB, C, H, W = x_ref.shape
    dct = dct_ref[...]
    band = band_ref[...]
    lane = jax.lax.broadcasted_iota(jnp.int32, (128,), 0)
    zr = jnp.zeros((3, W), jnp.float32)
    zc = jnp.zeros((H + 6, 3), jnp.float32)

    for b in range(BB):
        x = x_ref[b]

        prod = x * dct
        y = jnp.sum(jnp.sum(prod, axis=2), axis=1)[None, :]
        h = jnp.maximum(jnp.dot(y, w1_ref[...], preferred_element_type=jnp.float32), 0.0)
        att = jax.nn.sigmoid(jnp.dot(h, w2_ref[...], preferred_element_type=jnp.float32))
        att_ref[b] = att

        xs = x * att[0][:, None, None]
        mx = jnp.max(xs, axis=0)
        mn = jnp.sum(xs, axis=0) * (1.0 / C)

        cols = []
        for p in (mx, mn):
            p = jnp.concatenate([zr, p, zr], axis=0)
            p = jnp.concatenate([zc, p, zc], axis=1)
            for dx in range(7):
                cols.append(p[:, dx:dx + W])
        cols = jnp.concatenate(cols, axis=0)
        acc = jnp.dot(band, cols,
                      preferred_element_type=jnp.float32) + wb_ref[98]
        conv_ref[b] = acc

        s1 = jnp.sum(acc)
        s2 = jnp.sum(acc * acc)
        parts_ref[b, 0] = (jnp.where(lane == 0, s1, 0.0)
                           + jnp.where(lane == 1, s2, 0.0))


def _apply_kernel(x_ref, att_ref, conv_ref, parts_ref, out_ref):
    n = pl.program_id(0)
    BB, C, H, W = x_ref.shape
    N = conv_ref.shape[0]
    numel = N * H * W

    lane = jax.lax.broadcasted_iota(jnp.int32, (128,), 0)
    tot = jnp.sum(parts_ref[:, 0, :], axis=0)
    s1 = jnp.sum(jnp.where(lane == 0, tot, 0.0))
    s2 = jnp.sum(jnp.where(lane == 1, tot, 0.0))
    mean = s1 * (1.0 / numel)
    var = (s2 - s1 * mean) * (1.0 / (numel - 1))
    inv_sigma = 1.0 / (jnp.sqrt(2.0 * jnp.pi) * jnp.sqrt(var))

    for b in range(BB):
        nb = n * BB + b
        d = conv_ref[nb] - mean
        scale = jnp.exp(-(d * d) / (2.0 * var)) * inv_sigma
        for c in range(C):
            out_ref[b, c] = x_ref[b, c] * (att_ref[nb, c] * scale)


def kernel(x, dct_w, w1, w2, conv_wb):
    N, C, H, W = x.shape
    Cr = w1.shape[1]
    BB = _BB

    wt = conv_wb[:98].reshape(2, 7, 7)
    eyes = jnp.stack([jnp.eye(H, H + 6, k=dy, dtype=jnp.float32)
                      for dy in range(7)])
    blocks = [jnp.einsum("y,yij->ij", wt[c, :, dx], eyes)
              for c in range(2) for dx in range(7)]
    band = jnp.concatenate(blocks, axis=1)

    att3, conv, parts = pl.pallas_call(
        _gate_pool_conv_kernel,
        grid=(N // BB,),
        in_specs=[
            pl.BlockSpec((BB, C, H, W), lambda n: (n, 0, 0, 0)),
            pl.BlockSpec((C, H, W), lambda n: (0, 0, 0)),
            pl.BlockSpec((C, Cr), lambda n: (0, 0)),
            pl.BlockSpec((Cr, C), lambda n: (0, 0)),
            pl.BlockSpec((H, 14 * (H + 6)), lambda n: (0, 0)),
            pl.BlockSpec(memory_space=pltpu.MemorySpace.SMEM),
        ],
        out_specs=(
            pl.BlockSpec((BB, 1, C), lambda n: (n, 0, 0)),
            pl.BlockSpec((BB, H, W), lambda n: (n, 0, 0)),
            pl.BlockSpec((BB, 1, 128), lambda n: (n, 0, 0)),
        ),
        out_shape=(
            jax.ShapeDtypeStruct((N, 1, C), jnp.float32),
            jax.ShapeDtypeStruct((N, H, W), jnp.float32),
            jax.ShapeDtypeStruct((N, 1, 128), jnp.float32),
        ),
        compiler_params=pltpu.CompilerParams(dimension_semantics=("parallel",)),
    )(x, dct_w, w1, w2, band, conv_wb)

    out = pl.pallas_call(
        _apply_kernel,
        grid=(N // BB,),
        in_specs=[
            pl.BlockSpec((BB, C, H, W), lambda n: (n, 0, 0, 0)),
            pl.BlockSpec(memory_space=pltpu.MemorySpace.SMEM),
            pl.BlockSpec((N, H, W), lambda n: (0, 0, 0)),
            pl.BlockSpec((N, 1, 128), lambda n: (0, 0, 0)),
        ],
        out_specs=pl.BlockSpec((BB, C, H, W), lambda n: (n, 0, 0, 0)),
        out_shape=jax.ShapeDtypeStruct((N, C, H, W), jnp.float32),
        compiler_params=pltpu.CompilerParams(dimension_semantics=("parallel",)),
    )(x, att3.reshape(N, C), conv, parts)

    return out


# speedup vs baseline: 1.7711x; 1.6307x over previous
import jax
import jax.numpy as jnp
from jax.experimental import pallas as pl
from jax.experimental.pallas import tpu as pltpu

_BB = 8


def _gate_pool_kernel(x_ref, dct_ref, w1_ref, w2_ref, att_ref, pool_ref):
    BB, C, _ = x_ref.shape
    dct = dct_ref[...]
    for b in range(BB):
        x = x_ref[b]
        y = jnp.sum(x * dct, axis=1)[None, :]
        h = jnp.maximum(jnp.dot(y, w1_ref[...], preferred_element_type=jnp.float32), 0.0)
        att = jax.nn.sigmoid(jnp.dot(h, w2_ref[...], preferred_element_type=jnp.float32))
        att_ref[b] = att

        mx = jnp.max(x * att[0][:, None], axis=0)
        mn = jnp.dot(att, x, preferred_element_type=jnp.float32)[0] * (1.0 / C)
        pool_ref[b, 0] = mx
        pool_ref[b, 1] = mn


def _conv_kernel(pool_ref, band_ref, wb_ref, conv_ref, parts_ref):
    BB, _, H, W = pool_ref.shape
    band = band_ref[...]
    lane = jax.lax.broadcasted_iota(jnp.int32, (128,), 0)
    zr = jnp.zeros((3, W), jnp.float32)
    zc = jnp.zeros((H + 6, 3), jnp.float32)
    for b in range(BB):
        cols = []
        for c in range(2):
            p = jnp.concatenate([zr, pool_ref[b, c], zr], axis=0)
            p = jnp.concatenate([zc, p, zc], axis=1)
            for dx in range(7):
                cols.append(p[:, dx:dx + W])
        cols = jnp.concatenate(cols, axis=0)
        acc = jnp.dot(band, cols,
                      preferred_element_type=jnp.float32) + wb_ref[98]
        conv_ref[b] = acc
        s1 = jnp.sum(acc)
        s2 = jnp.sum(acc * acc)
        parts_ref[b, 0] = (jnp.where(lane == 0, s1, 0.0)
                           + jnp.where(lane == 1, s2, 0.0))


def _apply_kernel(x_ref, att_ref, convd_ref, parts_ref, out_ref):
    BB, C, HW = x_ref.shape
    N = parts_ref.shape[0]
    numel = N * HW

    lane = jax.lax.broadcasted_iota(jnp.int32, (128,), 0)
    tot = jnp.sum(parts_ref[:, 0, :], axis=0)
    s1 = jnp.sum(jnp.where(lane == 0, tot, 0.0))
    s2 = jnp.sum(jnp.where(lane == 1, tot, 0.0))
    mean = s1 * (1.0 / numel)
    var = (s2 - s1 * mean) * (1.0 / (numel - 1))
    inv_sigma = 1.0 / (jnp.sqrt(2.0 * jnp.pi) * jnp.sqrt(var))

    for b in range(BB):
        d = convd_ref[b] - mean
        scale = (jnp.exp(-(d * d) / (2.0 * var)) * inv_sigma)[None, :]
        att_col = att_ref[b][:, None]
        gate = jnp.dot(att_col, scale,
                       preferred_element_type=jnp.float32)
        out_ref[b] = x_ref[b] * gate


def kernel(x, dct_w, w1, w2, conv_wb):
    N, C, H, W = x.shape
    HW = H * W
    Cr = w1.shape[1]
    BB = _BB

    x2 = x.reshape(N, C, HW)
    dct2 = dct_w.reshape(C, HW)

    wt = conv_wb[:98].reshape(2, 7, 7)
    eyes = jnp.stack([jnp.eye(H, H + 6, k=dy, dtype=jnp.float32)
                      for dy in range(7)])
    blocks = [jnp.einsum("y,yij->ij", wt[c, :, dx], eyes)
              for c in range(2) for dx in range(7)]
    band = jnp.concatenate(blocks, axis=1)

    att3, pool = pl.pallas_call(
        _gate_pool_kernel,
        grid=(N // BB,),
        in_specs=[
            pl.BlockSpec((BB, C, HW), lambda n: (n, 0, 0)),
            pl.BlockSpec((C, HW), lambda n: (0, 0)),
            pl.BlockSpec((C, Cr), lambda n: (0, 0)),
            pl.BlockSpec((Cr, C), lambda n: (0, 0)),
        ],
        out_specs=(
            pl.BlockSpec((BB, 1, C), lambda n: (n, 0, 0)),
            pl.BlockSpec((BB, 2, HW), lambda n: (n, 0, 0)),
        ),
        out_shape=(
            jax.ShapeDtypeStruct((N, 1, C), jnp.float32),
            jax.ShapeDtypeStruct((N, 2, HW), jnp.float32),
        ),
        compiler_params=pltpu.CompilerParams(dimension_semantics=("parallel",)),
    )(x2, dct2, w1, w2)

    conv, parts = pl.pallas_call(
        _conv_kernel,
        grid=(N // BB,),
        in_specs=[
            pl.BlockSpec((BB, 2, H, W), lambda n: (n, 0, 0, 0)),
            pl.BlockSpec((H, 14 * (H + 6)), lambda n: (0, 0)),
            pl.BlockSpec(memory_space=pltpu.MemorySpace.SMEM),
        ],
        out_specs=(
            pl.BlockSpec((BB, H, W), lambda n: (n, 0, 0)),
            pl.BlockSpec((BB, 1, 128), lambda n: (n, 0, 0)),
        ),
        out_shape=(
            jax.ShapeDtypeStruct((N, H, W), jnp.float32),
            jax.ShapeDtypeStruct((N, 1, 128), jnp.float32),
        ),
        compiler_params=pltpu.CompilerParams(dimension_semantics=("parallel",)),
    )(pool.reshape(N, 2, H, W), band, conv_wb)

    out_flat = pl.pallas_call(
        _apply_kernel,
        grid=(N // BB,),
        in_specs=[
            pl.BlockSpec((BB, C, HW), lambda n: (n, 0, 0)),
            pl.BlockSpec((BB, C), lambda n: (n, 0)),
            pl.BlockSpec((BB, HW), lambda n: (n, 0)),
            pl.BlockSpec((N, 1, 128), lambda n: (0, 0, 0)),
        ],
        out_specs=pl.BlockSpec((BB, C, HW), lambda n: (n, 0, 0)),
        out_shape=jax.ShapeDtypeStruct((N, C, HW), jnp.float32),
        compiler_params=pltpu.CompilerParams(dimension_semantics=("parallel",)),
    )(x2, att3.reshape(N, C), conv.reshape(N, HW), parts)

    return out_flat.reshape(N, C, H, W)
